# Initial kernel scaffold; baseline (speedup 1.0000x reference)
import jax
import jax.numpy as jnp
from jax.experimental import pallas as pl


def kernel(drug_entity_indices, adj_entity, adj_relation, edge_weights, entity_emb, W0, res_w0, res_b0, W1, res_w1, res_b1, proj_w0, proj_b0, proj_w1, proj_b1, ln_g0, ln_b0, ln_g1, ln_b1, attn_in_w, attn_in_b, attn_out_w, attn_out_b, fn_g, fn_b):
    raise NotImplementedError("write your pallas kernel here")


# speedup vs baseline: 5.0919x; 5.0919x over previous
import functools

import jax
import jax.numpy as jnp
from jax import lax
from jax.experimental import pallas as pl
from jax.experimental.pallas import tpu as pltpu
from jax.experimental.pallas import tpu_sc as plsc

B = 4096
S = 32
D = 128
R = 4
H = 4
DH = D // H
EPS = 1e-5

_CHUNK = 128


def _sc_gather(table, idx_all):
    info = plsc.get_sparse_core_info()
    nc, ns = info.num_cores, info.num_subcores
    nw = nc * ns
    n = idx_all.shape[0]
    per_w = n // nw
    n_chunks = per_w // _CHUNK
    mesh = plsc.VectorSubcoreMesh(core_axis_name="c", subcore_axis_name="s")

    @functools.partial(
        pl.kernel,
        mesh=mesh,
        out_type=jax.ShapeDtypeStruct((n, D), jnp.float32),
        scratch_types=[
            pltpu.VMEM((_CHUNK,), jnp.int32),
            pltpu.VMEM((_CHUNK, D), jnp.float32),
            pltpu.SemaphoreType.DMA,
        ],
    )
    def k(table_hbm, idx_hbm, out_hbm, idx_v, rows_v, sem):
        wid = lax.axis_index("s") * nc + lax.axis_index("c")
        base_w = wid * per_w

        def chunk(i, carry):
            base = base_w + i * _CHUNK
            pltpu.sync_copy(idx_hbm.at[pl.ds(base, _CHUNK)], idx_v)
            pltpu.async_copy(table_hbm.at[idx_v], rows_v, sem).wait()
            pltpu.sync_copy(rows_v, out_hbm.at[pl.ds(base, _CHUNK)])
            return carry

        lax.fori_loop(0, n_chunks, chunk, 0)

    return k(table, idx_all)


_NBLK = 512


def _dot(a, b):
    return jnp.dot(a, b, preferred_element_type=jnp.float32)


def _ln(x, g, b):
    mu = jnp.mean(x, axis=-1, keepdims=True)
    var = jnp.mean((x - mu) ** 2, axis=-1, keepdims=True)
    return (x - mu) * lax.rsqrt(var + EPS) * g + b


def _tc_body(nb_ref, ew_ref, rel_ref, comb_ref, w0_ref, w1_ref, rw0_ref,
             rb0_ref, rw1_ref, rb1_ref, pw0_ref, pb0_ref, pw1_ref, pb1_ref,
             lg0_ref, lb0_ref, lg1_ref, lb1_ref, aw_ref, ab_ref, ow_ref,
             ob_ref, fg_ref, fb_ref, out_ref):
    nb = nb_ref[...].reshape(_NBLK, S, D)
    ew = ew_ref[...]
    rel = rel_ref[...]
    x = comb_ref[...]

    aggs = [
        jnp.sum(nb * jnp.where(rel == r, ew, 0.0)[:, :, None], axis=1)
        for r in range(R)
    ]
    msgs0 = sum(_dot(aggs[r], w0_ref[r]) for r in range(R))
    msgs1 = sum(_dot(aggs[r], w1_ref[r]) for r in range(R))

    h0 = jnp.maximum(x + msgs0 + _dot(x, rw0_ref[...].T) + rb0_ref[...], 0.0)
    n0 = _ln(_dot(h0, pw0_ref[...].T) + pb0_ref[...], lg0_ref[...], lb0_ref[...])
    h1 = jnp.maximum(h0 + msgs1 + _dot(h0, rw1_ref[...].T) + rb1_ref[...], 0.0)
    n1 = _ln(_dot(h1, pw1_ref[...].T) + pb1_ref[...], lg1_ref[...], lb1_ref[...])

    row = lax.broadcasted_iota(jnp.int32, (D, H), 0) // DH
    col = lax.broadcasted_iota(jnp.int32, (D, H), 1)
    eseg = (row == col).astype(jnp.float32)

    aw_t = aw_ref[...].T
    ab = ab_ref[...]
    qkv0 = _dot(n0, aw_t) + ab
    qkv1 = _dot(n1, aw_t) + ab
    scale = DH ** -0.5
    q0 = qkv0[:, :D] * scale
    k0 = qkv0[:, D:2 * D]
    v0 = qkv0[:, 2 * D:]
    q1 = qkv1[:, :D] * scale
    k1 = qkv1[:, D:2 * D]
    v1 = qkv1[:, 2 * D:]

    s00 = _dot(q0 * k0, eseg)
    s01 = _dot(q0 * k1, eseg)
    s10 = _dot(q1 * k0, eseg)
    s11 = _dot(q1 * k1, eseg)

    def softmax2(sa, sb):
        m = jnp.maximum(sa, sb)
        ea = jnp.exp(sa - m)
        eb = jnp.exp(sb - m)
        den = ea + eb
        return ea / den, eb / den

    a00, a01 = softmax2(s00, s01)
    a10, a11 = softmax2(s10, s11)
    o0 = _dot(a00, eseg.T) * v0 + _dot(a01, eseg.T) * v1
    o1 = _dot(a10, eseg.T) * v0 + _dot(a11, eseg.T) * v1

    ow_t = ow_ref[...].T
    ob = ob_ref[...]
    ao0 = _dot(o0, ow_t) + ob
    ao1 = _dot(o1, ow_t) + ob
    fused = 0.5 * (ao0 + ao1)
    out_ref[...] = _ln(fused, fg_ref[...], fb_ref[...])


def _tc_fused(nb_flat, ew, rel, combined, w0, w1, rw0, rb0, rw1, rb1, pw0,
              pb0, pw1, pb1, lg0, lb0, lg1, lb1, aw, ab, ow, ob, fg, fb,
              interpret=False):
    grid = (B // _NBLK,)

    def blk(shape):
        return pl.BlockSpec(shape, lambda i: (0,) * len(shape))

    specs = [
        pl.BlockSpec((_NBLK * S, D), lambda i: (i, 0)),
        pl.BlockSpec((_NBLK, S), lambda i: (i, 0)),
        pl.BlockSpec((_NBLK, S), lambda i: (i, 0)),
        pl.BlockSpec((_NBLK, D), lambda i: (i, 0)),
        blk((R, D, D)), blk((R, D, D)),
        blk((D, D)), blk((1, D)), blk((D, D)), blk((1, D)),
        blk((D, D)), blk((1, D)), blk((D, D)), blk((1, D)),
        blk((1, D)), blk((1, D)), blk((1, D)), blk((1, D)),
        blk((3 * D, D)), blk((1, 3 * D)),
        blk((D, D)), blk((1, D)),
        blk((1, D)), blk((1, D)),
    ]
    return pl.pallas_call(
        _tc_body,
        grid=grid,
        in_specs=specs,
        out_specs=pl.BlockSpec((_NBLK, D), lambda i: (i, 0)),
        out_shape=jax.ShapeDtypeStruct((B, D), jnp.float32),
        interpret=interpret,
    )(nb_flat, ew, rel, combined, w0, w1, rw0, rb0.reshape(1, D), rw1,
      rb1.reshape(1, D), pw0, pb0.reshape(1, D), pw1, pb1.reshape(1, D),
      lg0.reshape(1, D), lb0.reshape(1, D), lg1.reshape(1, D),
      lb1.reshape(1, D), aw, ab.reshape(1, 3 * D), ow, ob.reshape(1, D),
      fg.reshape(1, D), fb.reshape(1, D))


def kernel(drug_entity_indices, adj_entity, adj_relation, edge_weights,
           entity_emb, W0, res_w0, res_b0, W1, res_w1, res_b1, proj_w0,
           proj_b0, proj_w1, proj_b1, ln_g0, ln_b0, ln_g1, ln_b1, attn_in_w,
           attn_in_b, attn_out_w, attn_out_b, fn_g, fn_b):
    idx_all = jnp.concatenate(
        [adj_entity.reshape(-1), drug_entity_indices]).astype(jnp.int32)
    gathered = _sc_gather(entity_emb, idx_all)
    nb_flat = gathered[:B * S]
    combined = gathered[B * S:]
    return _tc_fused(nb_flat, edge_weights, adj_relation, combined, W0, W1,
                     res_w0, res_b0, res_w1, res_b1, proj_w0, proj_b0,
                     proj_w1, proj_b1, ln_g0, ln_b0, ln_g1, ln_b1, attn_in_w,
                     attn_in_b, attn_out_w, attn_out_b, fn_g, fn_b)


# speedup vs baseline: 6.0884x; 1.1957x over previous
import functools

import jax
import jax.numpy as jnp
from jax import lax
from jax.experimental import pallas as pl
from jax.experimental.pallas import tpu as pltpu
from jax.experimental.pallas import tpu_sc as plsc

B = 4096
S = 32
D = 128
R = 4
H = 4
DH = D // H
EPS = 1e-5

_C = 128
_NE = B * S


def _sc_agg(table, idx2, ew2, dst2, didx2):
    info = plsc.get_sparse_core_info()
    nc, ns = info.num_cores, info.num_subcores
    nw = nc * ns
    drugs_w = B // nw
    edges_w = _NE // nw
    rows_w = edges_w // _C
    n_pairs = rows_w // 2
    arows = R * drugs_w
    mesh = plsc.VectorSubcoreMesh(core_axis_name="c", subcore_axis_name="s")

    @functools.partial(
        pl.kernel,
        mesh=mesh,
        out_type=(
            jax.ShapeDtypeStruct((R, B, D), jnp.float32),
            jax.ShapeDtypeStruct((B, D), jnp.float32),
        ),
        scratch_types=[
            pltpu.VMEM((rows_w, _C), jnp.int32),
            pltpu.VMEM((rows_w, _C), jnp.float32),
            pltpu.VMEM((rows_w, _C), jnp.int32),
            pltpu.VMEM((_C,), jnp.int32),
            pltpu.VMEM((_C, D), jnp.float32),
            pltpu.VMEM((_C, D), jnp.float32),
            pltpu.VMEM((arows, D), jnp.float32),
            pltpu.SemaphoreType.DMA,
            pltpu.SemaphoreType.DMA,
            pltpu.SemaphoreType.DMA,
        ],
    )
    def k(table_hbm, idx_hbm, ew_hbm, dst_hbm, didx_hbm, agg_hbm, comb_hbm,
          idx_v, ew_v, dst_v, cidx_v, rows0, rows1, agg_v, gsem0, gsem1,
          wsem):
        wid = lax.axis_index("s") * nc + lax.axis_index("c")
        row0 = wid * rows_w

        pltpu.sync_copy(idx_hbm.at[pl.ds(row0, rows_w)], idx_v)
        pltpu.sync_copy(ew_hbm.at[pl.ds(row0, rows_w)], ew_v)
        pltpu.sync_copy(dst_hbm.at[pl.ds(row0, rows_w)], dst_v)

        pltpu.async_copy(table_hbm.at[idx_v.at[0]], rows0, gsem0)

        zero = jnp.zeros((16,), jnp.float32)

        def zrow(i, c):
            for u in range(8):
                agg_v[i, pl.ds(u * 16, 16)] = zero
            return c

        lax.fori_loop(0, arows, zrow, 0)

        def accum(i, buf):
            def grp(g, c):
                e0 = g * 16
                wv = ew_v[i, pl.ds(e0, 16)]
                dv = dst_v[i, pl.ds(e0, 16)]
                for l in range(16):
                    w = wv[l]
                    dr = dv[l]
                    for c8 in range(8):
                        v = buf[e0 + l, pl.ds(c8 * 16, 16)]
                        plsc.addupdate(agg_v.at[dr, pl.ds(c8 * 16, 16)],
                                       v * w)
                return c

            lax.fori_loop(0, _C // 16, grp, 0)

        def gwait(buf, sem):
            pltpu.make_async_copy(table_hbm.at[idx_v.at[0]], buf, sem).wait()

        def pair(t, c):
            i0 = 2 * t
            pltpu.async_copy(table_hbm.at[idx_v.at[i0 + 1]], rows1, gsem1)
            gwait(rows0, gsem0)
            accum(i0, rows0)

            @pl.when(t + 1 < n_pairs)
            def _():
                pltpu.async_copy(table_hbm.at[idx_v.at[i0 + 2]], rows0, gsem0)

            gwait(rows1, gsem1)
            accum(i0 + 1, rows1)
            return c

        lax.fori_loop(0, n_pairs, pair, 0)

        pltpu.sync_copy(didx_hbm.at[pl.ds(wid * drugs_w, drugs_w)], cidx_v)
        pltpu.async_copy(table_hbm.at[cidx_v], rows0, gsem0).wait()
        pltpu.async_copy(rows0, comb_hbm.at[pl.ds(wid * drugs_w, drugs_w)],
                         wsem)

        for r in range(R):
            pltpu.async_copy(
                agg_v.at[pl.ds(r * drugs_w, drugs_w)],
                agg_hbm.at[r, pl.ds(wid * drugs_w, drugs_w)], wsem)

        for _ in range(R + 1):
            pltpu.make_async_copy(rows0, comb_hbm.at[pl.ds(0, drugs_w)],
                                  wsem).wait()

    return k(table, idx2, ew2, dst2, didx2)


_NBLK = 512


def _dot(a, b):
    return jnp.dot(a, b, preferred_element_type=jnp.float32)


def _ln(x, g, b):
    mu = jnp.mean(x, axis=-1, keepdims=True)
    var = jnp.mean((x - mu) ** 2, axis=-1, keepdims=True)
    return (x - mu) * lax.rsqrt(var + EPS) * g + b


def _tc_body(agg_ref, comb_ref, w0_ref, w1_ref, rw0_ref,
             rb0_ref, rw1_ref, rb1_ref, pw0_ref, pb0_ref, pw1_ref, pb1_ref,
             lg0_ref, lb0_ref, lg1_ref, lb1_ref, aw_ref, ab_ref, ow_ref,
             ob_ref, fg_ref, fb_ref, out_ref):
    x = comb_ref[...]

    msgs0 = sum(_dot(agg_ref[r], w0_ref[r]) for r in range(R))
    msgs1 = sum(_dot(agg_ref[r], w1_ref[r]) for r in range(R))

    h0 = jnp.maximum(x + msgs0 + _dot(x, rw0_ref[...].T) + rb0_ref[...], 0.0)
    n0 = _ln(_dot(h0, pw0_ref[...].T) + pb0_ref[...], lg0_ref[...], lb0_ref[...])
    h1 = jnp.maximum(h0 + msgs1 + _dot(h0, rw1_ref[...].T) + rb1_ref[...], 0.0)
    n1 = _ln(_dot(h1, pw1_ref[...].T) + pb1_ref[...], lg1_ref[...], lb1_ref[...])

    row = lax.broadcasted_iota(jnp.int32, (D, H), 0) // DH
    col = lax.broadcasted_iota(jnp.int32, (D, H), 1)
    eseg = (row == col).astype(jnp.float32)

    aw_t = aw_ref[...].T
    ab = ab_ref[...]
    qkv0 = _dot(n0, aw_t) + ab
    qkv1 = _dot(n1, aw_t) + ab
    scale = DH ** -0.5
    q0 = qkv0[:, :D] * scale
    k0 = qkv0[:, D:2 * D]
    v0 = qkv0[:, 2 * D:]
    q1 = qkv1[:, :D] * scale
    k1 = qkv1[:, D:2 * D]
    v1 = qkv1[:, 2 * D:]

    s00 = _dot(q0 * k0, eseg)
    s01 = _dot(q0 * k1, eseg)
    s10 = _dot(q1 * k0, eseg)
    s11 = _dot(q1 * k1, eseg)

    def softmax2(sa, sb):
        m = jnp.maximum(sa, sb)
        ea = jnp.exp(sa - m)
        eb = jnp.exp(sb - m)
        den = ea + eb
        return ea / den, eb / den

    a00, a01 = softmax2(s00, s01)
    a10, a11 = softmax2(s10, s11)
    o0 = _dot(a00, eseg.T) * v0 + _dot(a01, eseg.T) * v1
    o1 = _dot(a10, eseg.T) * v0 + _dot(a11, eseg.T) * v1

    ow_t = ow_ref[...].T
    ob = ob_ref[...]
    ao0 = _dot(o0, ow_t) + ob
    ao1 = _dot(o1, ow_t) + ob
    fused = 0.5 * (ao0 + ao1)
    out_ref[...] = _ln(fused, fg_ref[...], fb_ref[...])


def _tc_fused(agg, combined, w0, w1, rw0, rb0, rw1, rb1, pw0,
              pb0, pw1, pb1, lg0, lb0, lg1, lb1, aw, ab, ow, ob, fg, fb,
              interpret=False):
    grid = (B // _NBLK,)

    def blk(shape):
        return pl.BlockSpec(shape, lambda i: (0,) * len(shape))

    specs = [
        pl.BlockSpec((R, _NBLK, D), lambda i: (0, i, 0)),
        pl.BlockSpec((_NBLK, D), lambda i: (i, 0)),
        blk((R, D, D)), blk((R, D, D)),
        blk((D, D)), blk((1, D)), blk((D, D)), blk((1, D)),
        blk((D, D)), blk((1, D)), blk((D, D)), blk((1, D)),
        blk((1, D)), blk((1, D)), blk((1, D)), blk((1, D)),
        blk((3 * D, D)), blk((1, 3 * D)),
        blk((D, D)), blk((1, D)),
        blk((1, D)), blk((1, D)),
    ]
    return pl.pallas_call(
        _tc_body,
        grid=grid,
        in_specs=specs,
        out_specs=pl.BlockSpec((_NBLK, D), lambda i: (i, 0)),
        out_shape=jax.ShapeDtypeStruct((B, D), jnp.float32),
        interpret=interpret,
    )(agg, combined, w0, w1, rw0, rb0.reshape(1, D), rw1,
      rb1.reshape(1, D), pw0, pb0.reshape(1, D), pw1, pb1.reshape(1, D),
      lg0.reshape(1, D), lb0.reshape(1, D), lg1.reshape(1, D),
      lb1.reshape(1, D), aw, ab.reshape(1, 3 * D), ow, ob.reshape(1, D),
      fg.reshape(1, D), fb.reshape(1, D))


def kernel(drug_entity_indices, adj_entity, adj_relation, edge_weights,
           entity_emb, W0, res_w0, res_b0, W1, res_w1, res_b1, proj_w0,
           proj_b0, proj_w1, proj_b1, ln_g0, ln_b0, ln_g1, ln_b1, attn_in_w,
           attn_in_b, attn_out_w, attn_out_b, fn_g, fn_b):
    nrows = _NE // _C
    drug_of_edge = lax.broadcasted_iota(jnp.int32, (B, S), 0)
    dst_local = adj_relation.astype(jnp.int32) * 128 + (drug_of_edge % 128)
    idx2 = adj_entity.astype(jnp.int32).reshape(nrows, _C)
    ew2 = edge_weights.reshape(nrows, _C)
    dst2 = dst_local.reshape(nrows, _C)
    didx2 = drug_entity_indices.astype(jnp.int32)

    agg, combined = _sc_agg(entity_emb, idx2, ew2, dst2, didx2)
    return _tc_fused(agg, combined, W0, W1,
                     res_w0, res_b0, res_w1, res_b1, proj_w0, proj_b0,
                     proj_w1, proj_b1, ln_g0, ln_b0, ln_g1, ln_b1, attn_in_w,
                     attn_in_b, attn_out_w, attn_out_b, fn_g, fn_b)
